# baseline (speedup 1.0000x reference)
import functools

import numpy as np
import jax
import jax.numpy as jnp
from jax.experimental import pallas as pl
from jax.experimental.pallas import tpu as pltpu

_NUM_P = 1024
_TINY = np.float32(np.finfo(np.float32).tiny)
_LO_N = np.float32(-0.9999999403953552)
_SQRT2 = np.float32(1.4142135381698608)

_ERFINV_A = [3.43273939e-07, -3.5233877e-06, -4.39150654e-06, 0.00021858087,
             -0.00125372503, -0.00417768164, 0.246640727, 1.50140941]
_ERFINV_B = [0.000100950558, 0.00134934322, -0.00367342844, 0.00573950773,
             -0.0076224613, 0.00943887047, 1.00167406, 2.83297682]


def _np_threefry(k0, k1, x0, x1):
    def rotl(v, r):
        return ((v << np.uint32(r)) | (v >> np.uint32(32 - r))).astype(np.uint32)
    x0 = np.asarray(x0, np.uint32).copy()
    x1 = np.asarray(x1, np.uint32).copy()
    k0 = np.uint32(k0)
    k1 = np.uint32(k1)
    ks2 = np.uint32(k0 ^ k1 ^ np.uint32(0x1BD11BDA))
    ks = [k0, k1, ks2]
    rots = [13, 15, 26, 6, 17, 29, 16, 24]
    x0 = (x0 + k0).astype(np.uint32)
    x1 = (x1 + k1).astype(np.uint32)
    for g in range(5):
        for r in (rots[0:4] if g % 2 == 0 else rots[4:8]):
            x0 = (x0 + x1).astype(np.uint32)
            x1 = rotl(x1, r)
            x1 = (x1 ^ x0).astype(np.uint32)
        x0 = (x0 + ks[(g + 1) % 3]).astype(np.uint32)
        x1 = (x1 + ks[(g + 2) % 3] + np.uint32(g + 1)).astype(np.uint32)
    return x0, x1


def _np_fold_in(key, data):
    o0, o1 = _np_threefry(key[0], key[1], np.array([0], np.uint32),
                          np.array([data], np.uint32))
    return np.array([o0[0], o1[0]], np.uint32)


def _step_keys(T):
    base = np.array([0, 42], np.uint32)
    kn = np.stack([_np_fold_in(base, 2 * t) for t in range(T)])
    kr = np.stack([_np_fold_in(base, 2 * t + 1) for t in range(T)])
    return kn.astype(np.int64).astype(np.int32), kr.astype(np.int64).astype(np.int32)


def _rotl(x, r):
    return jax.lax.shift_left(x, np.int32(r)) | jax.lax.shift_right_logical(
        x, np.int32(32 - r))


def _hash(k0, k1, cnt):
    ks2 = k0 ^ k1 ^ np.int32(0x1BD11BDA)
    ks = (k0, k1, ks2)
    ra = (13, 15, 26, 6)
    rb = (17, 29, 16, 24)
    x0 = jnp.zeros_like(cnt) + k0
    x1 = cnt + k1
    for g in range(5):
        for r in (ra if g % 2 == 0 else rb):
            x0 = x0 + x1
            x1 = _rotl(x1, r)
            x1 = x1 ^ x0
        x0 = x0 + ks[(g + 1) % 3]
        x1 = x1 + ks[(g + 2) % 3] + np.int32(g + 1)
    return x0 ^ x1


def _bits_to_unit(bits):
    m = jax.lax.shift_right_logical(bits, np.int32(9)) | np.int32(0x3F800000)
    return jax.lax.bitcast_convert_type(m, jnp.float32) - np.float32(1.0)


def _erfinv(x):
    w = -jnp.log1p(-x * x)
    wa = w - np.float32(2.5)
    pa = jnp.full_like(x, np.float32(2.81022636e-08))
    for c in _ERFINV_A:
        pa = pa * wa + np.float32(c)
    wb = jnp.sqrt(w) - np.float32(3.0)
    pb = jnp.full_like(x, np.float32(-0.000200214257))
    for c in _ERFINV_B:
        pb = pb * wb + np.float32(c)
    return jnp.where(w < np.float32(5.0), pa, pb) * x


def _pf_kernel(kn_ref, kr_ref, z_ref, obs_ref, out_ref, parts, newp, wts, rws,
               *, P, D, T, PT, KC, PC):
    b = pl.program_id(0)

    parts[...] = jnp.broadcast_to(z_ref[0, 0, :][None, :], (P, D))
    wts[...] = jnp.full((P, 1), np.float32(1.0 / P), jnp.float32)

    iota_nc_p = jax.lax.broadcasted_iota(jnp.int32, (PC, D), 0)
    iota_nc_d = jax.lax.broadcasted_iota(jnp.int32, (PC, D), 1)
    cnt_nc = iota_nc_p * np.int32(D) + iota_nc_d

    iota_kk = jax.lax.broadcasted_iota(jnp.int32, (KC, PT), 0)
    iota_kp = jax.lax.broadcasted_iota(jnp.int32, (KC, PT), 1)
    iota_oh = jax.lax.broadcasted_iota(jnp.int32, (P, PT), 0)

    def step(t, _):
        kn0 = kn_ref[t, 0]
        kn1 = kn_ref[t, 1]
        kr0 = kr_ref[t, 0]
        kr1 = kr_ref[t, 1]

        nbase = b * np.int32(P * D)

        def noise_chunk(c, carry):
            p0 = c * PC
            cnt = nbase + p0 * np.int32(D) + cnt_nc
            f = _bits_to_unit(_hash(kn0, kn1, cnt))
            u = jnp.maximum(_LO_N, f * np.float32(2.0) + _LO_N)
            noise = _SQRT2 * _erfinv(u)
            parts[pl.ds(p0, PC), :] = (parts[pl.ds(p0, PC), :]
                                       + np.float32(0.1) * noise)
            return carry

        jax.lax.fori_loop(0, P // PC, noise_chunk, 0, unroll=False)

        pr = parts[...]
        obs_t = obs_ref[0, t, :][None, :]
        d2 = jnp.sum((pr - obs_t) ** 2, axis=1, keepdims=True)
        lik = jnp.exp(np.float32(-0.5) * d2) + np.float32(1e-8)
        w = wts[...] * lik + np.float32(1e-10)
        w = w / jnp.sum(w)
        wts[...] = w
        rws[...] = np.float32(1.0) / w

        cbase = b * np.int32(P * P)

        def ptile(pt, carry):
            p0 = pt * PT

            def kchunk(kc, acc):
                minv, mini = acc
                k0 = kc * KC
                cnt = cbase + (p0 + iota_kp) * np.int32(P) + (k0 + iota_kk)
                f = _bits_to_unit(_hash(kr0, kr1, cnt))
                u = jnp.maximum(_TINY, f + _TINY)
                tv = -jnp.log(u)
                rwc = rws[pl.ds(k0, KC), :]
                val = tv * rwc
                mv = jnp.min(val, axis=0, keepdims=True)
                cand = jnp.where(val == mv, k0 + iota_kk, np.int32(2**30))
                mi = jnp.min(cand, axis=0, keepdims=True)
                better = mv < minv
                return (jnp.where(better, mv, minv),
                        jnp.where(better, mi, mini))

            minv0 = jnp.full((1, PT), np.float32(np.inf), jnp.float32)
            mini0 = jnp.zeros((1, PT), jnp.int32)
            _, mini = jax.lax.fori_loop(0, P // KC, kchunk, (minv0, mini0),
                                        unroll=False)

            onehot = (iota_oh == mini).astype(jnp.float32)
            gathered = jax.lax.dot_general(
                onehot, pr, (((0,), (0,)), ((), ())),
                precision=jax.lax.Precision.HIGHEST,
                preferred_element_type=jnp.float32)
            newp[pl.ds(p0, PT), :] = gathered
            return carry

        jax.lax.fori_loop(0, P // PT, ptile, 0, unroll=False)
        parts[...] = newp[...]
        return _

    jax.lax.fori_loop(0, T, step, 0, unroll=False)
    out_ref[0, 0, :] = jnp.sum(parts[...], axis=0) * np.float32(1.0 / P)


def _build(B, D, T, P, interpret=False):
    PT = min(128, P)
    KC = min(256, P)
    PC = min(256, P)
    grid_spec = pltpu.PrefetchScalarGridSpec(
        num_scalar_prefetch=2,
        grid=(B,),
        in_specs=[
            pl.BlockSpec((1, 1, D), lambda b, *_: (b, 0, 0)),
            pl.BlockSpec((1, T, D), lambda b, *_: (b, 0, 0)),
        ],
        out_specs=pl.BlockSpec((1, 1, D), lambda b, *_: (b, 0, 0)),
        scratch_shapes=[
            pltpu.VMEM((P, D), jnp.float32),
            pltpu.VMEM((P, D), jnp.float32),
            pltpu.VMEM((P, 1), jnp.float32),
            pltpu.VMEM((P, 1), jnp.float32),
        ],
    )
    return pl.pallas_call(
        functools.partial(_pf_kernel, P=P, D=D, T=T, PT=PT, KC=KC, PC=PC),
        grid_spec=grid_spec,
        out_shape=jax.ShapeDtypeStruct((B, 1, D), jnp.float32),
        interpret=interpret,
    )


def _run(z, observation, P, interpret=False):
    B, D = z.shape
    T = observation.shape[2]
    kn, kr = _step_keys(T)
    obs_t = jnp.transpose(observation, (0, 2, 1))
    call = _build(B, D, T, P, interpret=interpret)
    out = call(jnp.asarray(kn), jnp.asarray(kr), z[:, None, :], obs_t)
    return out[:, 0, :]


def kernel(z, observation):
    return _run(z, observation, _NUM_P)
